# baseline (speedup 1.0000x reference)
import functools

import jax
import jax.numpy as jnp
import numpy as np
from jax.experimental import pallas as pl
from jax.experimental.pallas import tpu as pltpu

B, T, N, C = 64, 10, 12, 1024
K2 = 9
TN = T * N
ROWS = TN * K2
TP, NP = T + 4, N + 4
P = TP * NP
NCONV = 27
RATIOS = (1, 2)


def _dyn_kernel(pf_ref, wall_ref, bias_ref, r_ref, q_ref, wht_ref,
                dyn_ref, mad_ref, tbl_ref, vp_ref):
    b = pl.program_id(0)

    @pl.when(b == 0)
    def _init():
        tbl_ref[...] = jnp.zeros_like(tbl_ref)
        vp_ref[...] = jnp.zeros_like(vp_ref)

    x = pf_ref[...].reshape(TN, C)

    for t in range(T):
        tbl_ref[(t + 2) * NP + 2:(t + 2) * NP + 2 + N, :] = x[t * N:(t + 1) * N, :]

    v = jax.lax.dot_general(x, wall_ref[...], (((1,), (0,)), ((), ())),
                            preferred_element_type=jnp.float32)
    vp_ref[26:26 + TN, :] = v

    nrow = jax.lax.broadcasted_iota(jnp.int32, (TN, 1), 0) % N

    tbl = tbl_ref[...]
    ms_acc = None
    m2 = None
    scale2 = None
    for r_idx, r in enumerate(RATIOS):
        acc = jnp.broadcast_to(bias_ref[0:1, r_idx * NCONV:(r_idx + 1) * NCONV],
                               (TN, NCONV)).astype(jnp.float32)
        for k in range(K2):
            di = (k // 3 - 1) * r
            dj = (k % 3 - 1) * r
            s = di * N + dj
            c0 = (r_idx * K2 + k) * NCONV
            sl = vp_ref[26 + s:26 + s + TN, c0:c0 + NCONV]
            nv = nrow + dj
            m = (nv >= 0) & (nv < N)
            acc = acc + jnp.where(m, sl, 0.0)

        offs = acc[:, :2 * K2]
        logits = acc[:, 2 * K2:NCONV]
        lmax = jnp.max(logits, axis=1, keepdims=True)
        e = jnp.exp(logits - lmax)
        scale = e / jnp.sum(e, axis=1, keepdims=True)

        tt = (jax.lax.broadcasted_iota(jnp.int32, (TN, K2), 0) // N).astype(jnp.float32)
        nn = (jax.lax.broadcasted_iota(jnp.int32, (TN, K2), 0) % N).astype(jnp.float32)
        kk = jax.lax.broadcasted_iota(jnp.int32, (TN, K2), 1)
        tapx = ((kk // 3) - 1).astype(jnp.float32) * r
        tapy = ((kk % 3) - 1).astype(jnp.float32) * r
        pos_x = tt + r + tapx + offs[:, :K2]
        pos_y = nn + r + tapy + offs[:, K2:2 * K2]
        xmax = float(T + 2 * r - 1)
        ymax = float(N + 2 * r - 1)
        xl = jnp.clip(jnp.floor(pos_x), 0.0, xmax)
        xr = jnp.clip(jnp.floor(pos_x) + 1.0, 0.0, xmax)
        yl = jnp.clip(jnp.floor(pos_y), 0.0, ymax)
        yr = jnp.clip(jnp.floor(pos_y) + 1.0, 0.0, ymax)
        pxc = jnp.clip(pos_x, 0.0, xmax)
        pyc = jnp.clip(pos_y, 0.0, ymax)
        wxl = 1.0 - jnp.abs(pxc - xl)
        wxr = 1.0 - jnp.abs(pxc - xr)
        wyl = 1.0 - jnp.abs(pyc - yl)
        wyr = 1.0 - jnp.abs(pyc - yr)

        fs = 2 - r
        i16 = jax.lax.broadcasted_iota(jnp.int32, (TN, K2, 16), 2)
        xli = xl.astype(jnp.int32) + fs
        xri = xr.astype(jnp.int32) + fs
        yli = yl.astype(jnp.int32) + fs
        yri = yr.astype(jnp.int32) + fs
        ax3 = (wxl[:, :, None] * (i16 == xli[:, :, None]) +
               wxr[:, :, None] * (i16 == xri[:, :, None]))
        ay3 = (wyl[:, :, None] * (i16 == yli[:, :, None]) +
               wyr[:, :, None] * (i16 == yri[:, :, None]))
        ax = ax3.reshape(ROWS, 16)
        ay = ay3.reshape(ROWS, 16)

        axrep = jax.lax.dot_general(ax, r_ref[...], (((1,), (0,)), ((), ())),
                                    preferred_element_type=jnp.float32)
        aytil = jax.lax.dot_general(ay, q_ref[...], (((1,), (0,)), ((), ())),
                                    preferred_element_type=jnp.float32)
        mmat = axrep * aytil

        msr = jnp.sum(mmat.reshape(TN, K2, P) * scale[:, :, None], axis=1)
        ms_acc = msr if ms_acc is None else ms_acc + msr
        if r == 2:
            m2 = mmat

    mad = jax.lax.dot_general(m2, tbl, (((1,), (0,)), ((), ())),
                              preferred_element_type=jnp.float32)
    mad_ref[...] = mad.reshape(T, N, K2, 1, 1, C)

    ms = ms_acc * 0.5
    ftm = jax.lax.dot_general(ms, tbl, (((1,), (0,)), ((), ())),
                              preferred_element_type=jnp.float32)
    dyn = jax.lax.dot_general(ftm, wht_ref[...], (((1,), (0,)), ((), ())),
                              preferred_element_type=jnp.float32)
    dyn_ref[...] = dyn.reshape(T, N, 1, 1, C)


@functools.partial(jax.jit, static_argnames=())
def _run(pf_flat, wall, bias, rmat, qmat, wht):
    grid = (B,)
    out_shapes = (
        jax.ShapeDtypeStruct((T, N, B, 1, C), jnp.float32),
        jax.ShapeDtypeStruct((T, N, K2, B, 1, C), jnp.float32),
    )
    return pl.pallas_call(
        _dyn_kernel,
        grid=grid,
        in_specs=[
            pl.BlockSpec((T, N, 1, 1, C), lambda b: (0, 0, b, 0, 0)),
            pl.BlockSpec((C, 2 * K2 * NCONV), lambda b: (0, 0)),
            pl.BlockSpec((1, 2 * NCONV), lambda b: (0, 0)),
            pl.BlockSpec((16, P), lambda b: (0, 0)),
            pl.BlockSpec((16, P), lambda b: (0, 0)),
            pl.BlockSpec((C, C), lambda b: (0, 0)),
        ],
        out_specs=(
            pl.BlockSpec((T, N, 1, 1, C), lambda b: (0, 0, b, 0, 0)),
            pl.BlockSpec((T, N, K2, 1, 1, C), lambda b: (0, 0, 0, b, 0, 0)),
        ),
        out_shape=out_shapes,
        scratch_shapes=[
            pltpu.VMEM((P, C), jnp.float32),
            pltpu.VMEM((TN + 52, 2 * K2 * NCONV), jnp.float32),
        ],
        compiler_params=pltpu.CompilerParams(
            dimension_semantics=("arbitrary",),
        ),
    )(pf_flat, wall, bias, rmat, qmat, wht)


def kernel(person_features, W_hidden, Wp_1, bp_1, Ws_1, bs_1, Wp_2, bp_2, Ws_2, bs_2):
    pf_tn = person_features.transpose(1, 2, 0, 3).reshape(T, N, B, 1, C)

    walls = []
    biases = []
    for Wp, bp, Ws, bs in ((Wp_1, bp_1, Ws_1, bs_1), (Wp_2, bp_2, Ws_2, bs_2)):
        wcat = jnp.concatenate([Wp, Ws], axis=0)
        w = wcat.transpose(2, 3, 1, 0).reshape(K2, C, NCONV)
        walls.append(w.transpose(1, 0, 2).reshape(C, K2 * NCONV))
        biases.append(jnp.concatenate([bp, bs], axis=0))
    wall = jnp.concatenate(walls, axis=1)
    bias = jnp.concatenate(biases, axis=0).reshape(1, 2 * NCONV)

    pidx = np.arange(P)
    rmat = jnp.asarray((pidx[None, :] // NP) == np.arange(16)[:, None],
                       dtype=jnp.float32)
    qmat = jnp.asarray((pidx[None, :] % NP) == np.arange(16)[:, None],
                       dtype=jnp.float32)

    wht = W_hidden.T

    dyn_p, mad_p = _run(pf_tn, wall, bias, rmat, qmat, wht)
    dyn = dyn_p.reshape(T, N, B, C).transpose(2, 0, 1, 3)
    mad = mad_p.reshape(T, N, K2, B, C).transpose(3, 0, 1, 2, 4)
    return dyn, mad


# speedup vs baseline: 1.8170x; 1.8170x over previous
import functools

import jax
import jax.numpy as jnp
import numpy as np
from jax.experimental import pallas as pl
from jax.experimental.pallas import tpu as pltpu

B, T, N, C = 64, 10, 12, 1024
K2 = 9
TN = T * N
ROWS = TN * K2
TP, NP = T + 4, N + 4
P = TP * NP
PK = 256
NCONV = 27
RATIOS = (1, 2)
BF = jnp.bfloat16


def _dyn_kernel(pf_ref, wall_ref, bias_ref, ef_ref, sx_ref, sy_ref, whtb_ref,
                dyn_ref, mad_ref, tblb_ref, vp_ref):
    b = pl.program_id(0)

    @pl.when(b == 0)
    def _init():
        tblb_ref[...] = jnp.zeros_like(tblb_ref)
        vp_ref[...] = jnp.zeros_like(vp_ref)

    x = pf_ref[...].reshape(TN, C)
    xb = x.astype(BF)

    for t in range(T):
        tblb_ref[(t + 2) * NP + 2:(t + 2) * NP + 2 + N, :] = xb[t * N:(t + 1) * N, :]

    v = jax.lax.dot_general(x, wall_ref[...], (((1,), (0,)), ((), ())),
                            preferred_element_type=jnp.float32)
    vp_ref[26:26 + TN, :] = v

    nrow = jax.lax.broadcasted_iota(jnp.int32, (TN, 1), 0) % N
    i144 = (jax.lax.broadcasted_iota(jnp.int32, (1, 144), 1) % 16
            ).astype(jnp.float32)

    tblb = tblb_ref[...]
    ms_acc = None
    for r_idx, r in enumerate(RATIOS):
        acc = jnp.broadcast_to(bias_ref[0:1, r_idx * NCONV:(r_idx + 1) * NCONV],
                               (TN, NCONV)).astype(jnp.float32)
        for k in range(K2):
            di = (k // 3 - 1) * r
            dj = (k % 3 - 1) * r
            s = di * N + dj
            c0 = (r_idx * K2 + k) * NCONV
            sl = vp_ref[26 + s:26 + s + TN, c0:c0 + NCONV]
            nv = nrow + dj
            m = (nv >= 0) & (nv < N)
            acc = acc + jnp.where(m, sl, 0.0)

        offs = acc[:, :2 * K2]
        logits = acc[:, 2 * K2:NCONV]
        lmax = jnp.max(logits, axis=1, keepdims=True)
        e = jnp.exp(logits - lmax)
        scale = e / jnp.sum(e, axis=1, keepdims=True)

        tt = (jax.lax.broadcasted_iota(jnp.int32, (TN, K2), 0) // N).astype(jnp.float32)
        nn = (jax.lax.broadcasted_iota(jnp.int32, (TN, K2), 0) % N).astype(jnp.float32)
        kk = jax.lax.broadcasted_iota(jnp.int32, (TN, K2), 1)
        tapx = ((kk // 3) - 1).astype(jnp.float32) * r
        tapy = ((kk % 3) - 1).astype(jnp.float32) * r
        pos_x = tt + r + tapx + offs[:, :K2]
        pos_y = nn + r + tapy + offs[:, K2:2 * K2]
        xmax = float(T + 2 * r - 1)
        ymax = float(N + 2 * r - 1)
        xl = jnp.clip(jnp.floor(pos_x), 0.0, xmax)
        xr = jnp.clip(jnp.floor(pos_x) + 1.0, 0.0, xmax)
        yl = jnp.clip(jnp.floor(pos_y), 0.0, ymax)
        yr = jnp.clip(jnp.floor(pos_y) + 1.0, 0.0, ymax)
        pxc = jnp.clip(pos_x, 0.0, xmax)
        pyc = jnp.clip(pos_y, 0.0, ymax)
        fs = float(2 - r)
        fields = jnp.concatenate(
            [1.0 - jnp.abs(pxc - xl), 1.0 - jnp.abs(pxc - xr), xl + fs, xr + fs,
             1.0 - jnp.abs(pyc - yl), 1.0 - jnp.abs(pyc - yr), yl + fs, yr + fs],
            axis=0)

        bc = jax.lax.dot_general(fields, ef_ref[...], (((1,), (0,)), ((), ())),
                                 preferred_element_type=jnp.float32)
        axl = jnp.where(i144 == bc[2 * TN:3 * TN], bc[0:TN], 0.0) + \
            jnp.where(i144 == bc[3 * TN:4 * TN], bc[TN:2 * TN], 0.0)
        ayl = jnp.where(i144 == bc[6 * TN:7 * TN], bc[4 * TN:5 * TN], 0.0) + \
            jnp.where(i144 == bc[7 * TN:8 * TN], bc[5 * TN:6 * TN], 0.0)

        axv = jax.lax.dot_general(axl.astype(BF), sx_ref[...],
                                  (((1,), (0,)), ((), ())),
                                  preferred_element_type=jnp.float32)
        ayv = jax.lax.dot_general(ayl.astype(BF), sy_ref[...],
                                  (((1,), (0,)), ((), ())),
                                  preferred_element_type=jnp.float32)
        m2l = (axv * ayv).astype(BF)

        msr = None
        for k in range(K2):
            term = scale[:, k:k + 1] * m2l[:, k * PK:(k + 1) * PK].astype(jnp.float32)
            msr = term if msr is None else msr + term
        ms_acc = msr if ms_acc is None else ms_acc + msr

        if r == 2:
            for k in range(K2):
                madk = jax.lax.dot_general(
                    m2l[:, k * PK:(k + 1) * PK], tblb,
                    (((1,), (0,)), ((), ())),
                    preferred_element_type=jnp.float32)
                mad_ref[:, :, k, 0, 0, :] = madk.reshape(T, N, C)

    ms = (ms_acc * 0.5).astype(BF)
    ftm = jax.lax.dot_general(ms, tblb, (((1,), (0,)), ((), ())),
                              preferred_element_type=jnp.float32)
    dyn = jax.lax.dot_general(ftm.astype(BF), whtb_ref[...],
                              (((1,), (0,)), ((), ())),
                              preferred_element_type=jnp.float32)
    dyn_ref[...] = dyn.reshape(T, N, 1, 1, C)


@functools.partial(jax.jit, static_argnames=())
def _run(pf_tn, wall, bias, ef, sx, sy, whtb):
    grid = (B,)
    out_shapes = (
        jax.ShapeDtypeStruct((T, N, B, 1, C), jnp.float32),
        jax.ShapeDtypeStruct((T, N, K2, B, 1, C), jnp.float32),
    )
    return pl.pallas_call(
        _dyn_kernel,
        grid=grid,
        in_specs=[
            pl.BlockSpec((T, N, 1, 1, C), lambda b: (0, 0, b, 0, 0)),
            pl.BlockSpec((C, 2 * K2 * NCONV), lambda b: (0, 0)),
            pl.BlockSpec((1, 2 * NCONV), lambda b: (0, 0)),
            pl.BlockSpec((K2, 144), lambda b: (0, 0)),
            pl.BlockSpec((144, K2 * PK), lambda b: (0, 0)),
            pl.BlockSpec((144, K2 * PK), lambda b: (0, 0)),
            pl.BlockSpec((C, C), lambda b: (0, 0)),
        ],
        out_specs=(
            pl.BlockSpec((T, N, 1, 1, C), lambda b: (0, 0, b, 0, 0)),
            pl.BlockSpec((T, N, K2, 1, 1, C), lambda b: (0, 0, 0, b, 0, 0)),
        ),
        out_shape=out_shapes,
        scratch_shapes=[
            pltpu.VMEM((PK, C), BF),
            pltpu.VMEM((TN + 52, 2 * K2 * NCONV), jnp.float32),
        ],
        compiler_params=pltpu.CompilerParams(
            dimension_semantics=("arbitrary",),
        ),
    )(pf_tn, wall, bias, ef, sx, sy, whtb)


def kernel(person_features, W_hidden, Wp_1, bp_1, Ws_1, bs_1, Wp_2, bp_2, Ws_2, bs_2):
    pf_tn = person_features.transpose(1, 2, 0, 3).reshape(T, N, B, 1, C)

    walls = []
    biases = []
    for Wp, bp, Ws, bs in ((Wp_1, bp_1, Ws_1, bs_1), (Wp_2, bp_2, Ws_2, bs_2)):
        wcat = jnp.concatenate([Wp, Ws], axis=0)
        w = wcat.transpose(2, 3, 1, 0).reshape(K2, C, NCONV)
        walls.append(w.transpose(1, 0, 2).reshape(C, K2 * NCONV))
        biases.append(jnp.concatenate([bp, bs], axis=0))
    wall = jnp.concatenate(walls, axis=1)
    bias = jnp.concatenate(biases, axis=0).reshape(1, 2 * NCONV)

    k_ar = np.arange(K2)
    ef_np = np.zeros((K2, 144), np.float32)
    ef_np[np.repeat(k_ar, 16), np.arange(144)] = 1.0
    sx_np = np.zeros((144, K2 * PK), np.float32)
    sy_np = np.zeros((144, K2 * PK), np.float32)
    for k in range(K2):
        for xx in range(TP):
            for yy in range(NP):
                p = k * PK + xx * NP + yy
                sx_np[k * 16 + xx, p] = 1.0
                sy_np[k * 16 + yy, p] = 1.0
    ef = jnp.asarray(ef_np)
    sx = jnp.asarray(sx_np, dtype=BF)
    sy = jnp.asarray(sy_np, dtype=BF)

    whtb = W_hidden.T.astype(BF)

    dyn_p, mad_p = _run(pf_tn, wall, bias, ef, sx, sy, whtb)
    dyn = dyn_p.reshape(T, N, B, C).transpose(2, 0, 1, 3)
    mad = mad_p.reshape(T, N, K2, B, C).transpose(3, 0, 1, 2, 4)
    return dyn, mad


# speedup vs baseline: 1.9169x; 1.0550x over previous
import functools

import jax
import jax.numpy as jnp
import numpy as np
from jax.experimental import pallas as pl
from jax.experimental.pallas import tpu as pltpu

B, T, N, C = 64, 10, 12, 1024
K2 = 9
TN = T * N
ROWS = TN * K2
TP, NP = T + 4, N + 4
P = TP * NP
PK = 256
NCONV = 27
RATIOS = (1, 2)
BF = jnp.bfloat16


def _dyn_kernel(pf_ref, wall_ref, bias_ref, ef_ref, sx_ref, sy_ref, whtb_ref,
                dyn_ref, mad_ref, tblb_ref, vp_ref):
    b = pl.program_id(0)

    @pl.when(b == 0)
    def _init():
        tblb_ref[...] = jnp.zeros_like(tblb_ref)
        vp_ref[...] = jnp.zeros_like(vp_ref)

    x = pf_ref[0]
    xb = x.astype(BF)

    for t in range(T):
        tblb_ref[(t + 2) * NP + 2:(t + 2) * NP + 2 + N, :] = xb[t * N:(t + 1) * N, :]

    v = jax.lax.dot_general(x, wall_ref[...], (((1,), (0,)), ((), ())),
                            preferred_element_type=jnp.float32)
    vp_ref[26:26 + TN, :] = v

    nrow = jax.lax.broadcasted_iota(jnp.int32, (TN, 1), 0) % N
    i144 = (jax.lax.broadcasted_iota(jnp.int32, (1, 144), 1) % 16
            ).astype(BF)

    tblb = tblb_ref[...]
    ms_acc = None
    for r_idx, r in enumerate(RATIOS):
        acc = jnp.broadcast_to(bias_ref[0:1, r_idx * NCONV:(r_idx + 1) * NCONV],
                               (TN, NCONV)).astype(jnp.float32)
        for k in range(K2):
            di = (k // 3 - 1) * r
            dj = (k % 3 - 1) * r
            s = di * N + dj
            c0 = (r_idx * K2 + k) * NCONV
            sl = vp_ref[26 + s:26 + s + TN, c0:c0 + NCONV]
            nv = nrow + dj
            m = (nv >= 0) & (nv < N)
            acc = acc + jnp.where(m, sl, 0.0)

        offs = acc[:, :2 * K2]
        logits = acc[:, 2 * K2:NCONV]
        lmax = jnp.max(logits, axis=1, keepdims=True)
        e = jnp.exp(logits - lmax)
        scale = e / jnp.sum(e, axis=1, keepdims=True)

        tt = (jax.lax.broadcasted_iota(jnp.int32, (TN, K2), 0) // N).astype(jnp.float32)
        nn = (jax.lax.broadcasted_iota(jnp.int32, (TN, K2), 0) % N).astype(jnp.float32)
        kk = jax.lax.broadcasted_iota(jnp.int32, (TN, K2), 1)
        tapx = ((kk // 3) - 1).astype(jnp.float32) * r
        tapy = ((kk % 3) - 1).astype(jnp.float32) * r
        pos_x = tt + r + tapx + offs[:, :K2]
        pos_y = nn + r + tapy + offs[:, K2:2 * K2]
        xmax = float(T + 2 * r - 1)
        ymax = float(N + 2 * r - 1)
        xl = jnp.clip(jnp.floor(pos_x), 0.0, xmax)
        xr = jnp.clip(jnp.floor(pos_x) + 1.0, 0.0, xmax)
        yl = jnp.clip(jnp.floor(pos_y), 0.0, ymax)
        yr = jnp.clip(jnp.floor(pos_y) + 1.0, 0.0, ymax)
        pxc = jnp.clip(pos_x, 0.0, xmax)
        pyc = jnp.clip(pos_y, 0.0, ymax)
        fs = float(2 - r)
        fields = jnp.concatenate(
            [1.0 - jnp.abs(pxc - xl), 1.0 - jnp.abs(pxc - xr), xl + fs, xr + fs,
             1.0 - jnp.abs(pyc - yl), 1.0 - jnp.abs(pyc - yr), yl + fs, yr + fs],
            axis=0).astype(BF)

        bc = jax.lax.dot_general(fields, ef_ref[...], (((1,), (0,)), ((), ())),
                                 preferred_element_type=jnp.float32)
        bcb = bc.astype(BF)
        axl = jnp.where(i144 == bcb[2 * TN:3 * TN], bcb[0:TN], 0.0) + \
            jnp.where(i144 == bcb[3 * TN:4 * TN], bcb[TN:2 * TN], 0.0)
        ayl = jnp.where(i144 == bcb[6 * TN:7 * TN], bcb[4 * TN:5 * TN], 0.0) + \
            jnp.where(i144 == bcb[7 * TN:8 * TN], bcb[5 * TN:6 * TN], 0.0)

        axv = jax.lax.dot_general(axl, sx_ref[...],
                                  (((1,), (0,)), ((), ())),
                                  preferred_element_type=jnp.float32)
        ayv = jax.lax.dot_general(ayl, sy_ref[...],
                                  (((1,), (0,)), ((), ())),
                                  preferred_element_type=jnp.float32)
        m2l = (axv * ayv).astype(BF)

        msr = None
        for k in range(K2):
            term = scale[:, k:k + 1] * m2l[:, k * PK:(k + 1) * PK].astype(jnp.float32)
            msr = term if msr is None else msr + term
        ms_acc = msr if ms_acc is None else ms_acc + msr

        if r == 2:
            for k in range(K2):
                madk = jax.lax.dot_general(
                    m2l[:, k * PK:(k + 1) * PK], tblb,
                    (((1,), (0,)), ((), ())),
                    preferred_element_type=jnp.float32)
                mad_ref[:, :, k, 0, 0, :] = madk.reshape(T, N, C)

    ms = (ms_acc * 0.5).astype(BF)
    ftm = jax.lax.dot_general(ms, tblb, (((1,), (0,)), ((), ())),
                              preferred_element_type=jnp.float32)
    dyn = jax.lax.dot_general(ftm.astype(BF), whtb_ref[...],
                              (((1,), (0,)), ((), ())),
                              preferred_element_type=jnp.float32)
    dyn_ref[...] = dyn.reshape(T, N, 1, 1, C)


@functools.partial(jax.jit, static_argnames=())
def _run(pf_tn, wall, bias, ef, sx, sy, whtb):
    grid = (B,)
    out_shapes = (
        jax.ShapeDtypeStruct((T, N, B, 1, C), jnp.float32),
        jax.ShapeDtypeStruct((T, N, K2, B, 1, C), jnp.float32),
    )
    return pl.pallas_call(
        _dyn_kernel,
        grid=grid,
        in_specs=[
            pl.BlockSpec((1, TN, C), lambda b: (b, 0, 0)),
            pl.BlockSpec((C, 2 * K2 * NCONV), lambda b: (0, 0)),
            pl.BlockSpec((1, 2 * NCONV), lambda b: (0, 0)),
            pl.BlockSpec((K2, 144), lambda b: (0, 0)),
            pl.BlockSpec((144, K2 * PK), lambda b: (0, 0)),
            pl.BlockSpec((144, K2 * PK), lambda b: (0, 0)),
            pl.BlockSpec((C, C), lambda b: (0, 0)),
        ],
        out_specs=(
            pl.BlockSpec((T, N, 1, 1, C), lambda b: (0, 0, b, 0, 0)),
            pl.BlockSpec((T, N, K2, 1, 1, C), lambda b: (0, 0, 0, b, 0, 0)),
        ),
        out_shape=out_shapes,
        scratch_shapes=[
            pltpu.VMEM((PK, C), BF),
            pltpu.VMEM((TN + 52, 2 * K2 * NCONV), jnp.float32),
        ],
        compiler_params=pltpu.CompilerParams(
            dimension_semantics=("arbitrary",),
        ),
    )(pf_tn, wall, bias, ef, sx, sy, whtb)


def kernel(person_features, W_hidden, Wp_1, bp_1, Ws_1, bs_1, Wp_2, bp_2, Ws_2, bs_2):
    pf_tn = person_features.reshape(B, TN, C)

    walls = []
    biases = []
    for Wp, bp, Ws, bs in ((Wp_1, bp_1, Ws_1, bs_1), (Wp_2, bp_2, Ws_2, bs_2)):
        wcat = jnp.concatenate([Wp, Ws], axis=0)
        w = wcat.transpose(2, 3, 1, 0).reshape(K2, C, NCONV)
        walls.append(w.transpose(1, 0, 2).reshape(C, K2 * NCONV))
        biases.append(jnp.concatenate([bp, bs], axis=0))
    wall = jnp.concatenate(walls, axis=1)
    bias = jnp.concatenate(biases, axis=0).reshape(1, 2 * NCONV)

    k_ar = np.arange(K2)
    ef_np = np.zeros((K2, 144), np.float32)
    ef_np[np.repeat(k_ar, 16), np.arange(144)] = 1.0
    sx_np = np.zeros((144, K2 * PK), np.float32)
    sy_np = np.zeros((144, K2 * PK), np.float32)
    for k in range(K2):
        for xx in range(TP):
            for yy in range(NP):
                p = k * PK + xx * NP + yy
                sx_np[k * 16 + xx, p] = 1.0
                sy_np[k * 16 + yy, p] = 1.0
    ef = jnp.asarray(ef_np, dtype=BF)
    sx = jnp.asarray(sx_np, dtype=BF)
    sy = jnp.asarray(sy_np, dtype=BF)

    whtb = W_hidden.T.astype(BF)

    dyn_p, mad_p = _run(pf_tn, wall, bias, ef, sx, sy, whtb)
    dyn = dyn_p.reshape(T, N, B, C).transpose(2, 0, 1, 3)
    mad = mad_p.reshape(T, N, K2, B, C).transpose(3, 0, 1, 2, 4)
    return dyn, mad


# speedup vs baseline: 3.1782x; 1.6580x over previous
import functools

import jax
import jax.numpy as jnp
import numpy as np
from jax.experimental import pallas as pl
from jax.experimental.pallas import tpu as pltpu

B, T, N, C = 64, 10, 12, 1024
K2 = 9
TN = T * N
G = 8
RG = TN * G
TP, NP = T + 4, N + 4
P = TP * NP
PK = 256
NCONV = 27
MARG = 26 * G
RATIOS = (1, 2)
BF = jnp.bfloat16


def _dyn_kernel(pf_ref, wall_ref, bias_ref, ef_ref, sx_ref, sy_ref, whtb_ref,
                dyn_ref, mad_ref, m2g_ref, tblg_ref, vp_ref):
    g = pl.program_id(0)
    k = pl.program_id(1)

    @pl.when((g == 0) & (k == 0))
    def _init():
        tblg_ref[...] = jnp.zeros_like(tblg_ref)
        vp_ref[...] = jnp.zeros_like(vp_ref)

    @pl.when(k == 0)
    def _frontend():
        xg = pf_ref[...]
        xi = xg.reshape(RG, C)

        xc = xi.reshape(TN, G, C).transpose(1, 0, 2).astype(BF)
        for t in range(T):
            tblg_ref[:, (t + 2) * NP + 2:(t + 2) * NP + 2 + N, :] = \
                xc[:, t * N:(t + 1) * N, :]

        v = jax.lax.dot_general(xi, wall_ref[...], (((1,), (0,)), ((), ())),
                                preferred_element_type=jnp.float32)
        vp_ref[MARG:MARG + RG, :] = v

        nrow = (jax.lax.broadcasted_iota(jnp.int32, (RG, 1), 0) // G) % N
        i144 = (jax.lax.broadcasted_iota(jnp.int32, (1, 144), 1) % 16
                ).astype(BF)

        ms_acc = None
        for r_idx, r in enumerate(RATIOS):
            acc = jnp.broadcast_to(
                bias_ref[0:1, r_idx * NCONV:(r_idx + 1) * NCONV],
                (RG, NCONV)).astype(jnp.float32)
            for kk_ in range(K2):
                di = (kk_ // 3 - 1) * r
                dj = (kk_ % 3 - 1) * r
                s = (di * N + dj) * G
                c0 = (r_idx * K2 + kk_) * NCONV
                sl = vp_ref[MARG + s:MARG + s + RG, c0:c0 + NCONV]
                nv = nrow + dj
                m = (nv >= 0) & (nv < N)
                acc = acc + jnp.where(m, sl, 0.0)

            offs = acc[:, :2 * K2]
            logits = acc[:, 2 * K2:NCONV]
            lmax = jnp.max(logits, axis=1, keepdims=True)
            e = jnp.exp(logits - lmax)
            scale = e / jnp.sum(e, axis=1, keepdims=True)

            rho = jax.lax.broadcasted_iota(jnp.int32, (RG, K2), 0)
            tt = (rho // (N * G)).astype(jnp.float32)
            nn = ((rho // G) % N).astype(jnp.float32)
            kk = jax.lax.broadcasted_iota(jnp.int32, (RG, K2), 1)
            tapx = ((kk // 3) - 1).astype(jnp.float32) * r
            tapy = ((kk % 3) - 1).astype(jnp.float32) * r
            pos_x = tt + r + tapx + offs[:, :K2]
            pos_y = nn + r + tapy + offs[:, K2:2 * K2]
            xmax = float(T + 2 * r - 1)
            ymax = float(N + 2 * r - 1)
            xl = jnp.clip(jnp.floor(pos_x), 0.0, xmax)
            xr = jnp.clip(jnp.floor(pos_x) + 1.0, 0.0, xmax)
            yl = jnp.clip(jnp.floor(pos_y), 0.0, ymax)
            yr = jnp.clip(jnp.floor(pos_y) + 1.0, 0.0, ymax)
            pxc = jnp.clip(pos_x, 0.0, xmax)
            pyc = jnp.clip(pos_y, 0.0, ymax)
            fs = float(2 - r)
            fields = jnp.concatenate(
                [1.0 - jnp.abs(pxc - xl), 1.0 - jnp.abs(pxc - xr),
                 xl + fs, xr + fs,
                 1.0 - jnp.abs(pyc - yl), 1.0 - jnp.abs(pyc - yr),
                 yl + fs, yr + fs],
                axis=0).astype(BF)

            bc = jax.lax.dot_general(fields, ef_ref[...],
                                     (((1,), (0,)), ((), ())),
                                     preferred_element_type=jnp.float32)
            bcb = bc.astype(BF)
            axl = jnp.where(i144 == bcb[2 * RG:3 * RG], bcb[0:RG], 0.0) + \
                jnp.where(i144 == bcb[3 * RG:4 * RG], bcb[RG:2 * RG], 0.0)
            ayl = jnp.where(i144 == bcb[6 * RG:7 * RG], bcb[4 * RG:5 * RG], 0.0) + \
                jnp.where(i144 == bcb[7 * RG:8 * RG], bcb[5 * RG:6 * RG], 0.0)

            axv = jax.lax.dot_general(axl, sx_ref[...], (((1,), (0,)), ((), ())),
                                      preferred_element_type=jnp.float32)
            ayv = jax.lax.dot_general(ayl, sy_ref[...], (((1,), (0,)), ((), ())),
                                      preferred_element_type=jnp.float32)
            m2l = (axv * ayv).astype(BF)

            msr = None
            for kk_ in range(K2):
                term = scale[:, kk_:kk_ + 1] * \
                    m2l[:, kk_ * PK:(kk_ + 1) * PK].astype(jnp.float32)
                msr = term if msr is None else msr + term
            ms_acc = msr if ms_acc is None else ms_acc + msr

            if r == 2:
                m2g_ref[...] = m2l.reshape(TN, G, K2 * PK).transpose(1, 0, 2)

        msb = (ms_acc * 0.5).astype(BF).reshape(TN, G, PK).transpose(1, 0, 2)
        ftms = []
        for bb in range(G):
            ftms.append(jax.lax.dot_general(
                msb[bb], tblg_ref[bb], (((1,), (0,)), ((), ())),
                preferred_element_type=jnp.float32))
        ftmc = jnp.stack(ftms, axis=0)
        ftmi = ftmc.transpose(1, 0, 2).reshape(RG, C)
        dyn = jax.lax.dot_general(ftmi.astype(BF), whtb_ref[...],
                                  (((1,), (0,)), ((), ())),
                                  preferred_element_type=jnp.float32)
        dyn_ref[...] = dyn.reshape(T, N, 1, G, C)

    mads = []
    for bb in range(G):
        m2k = m2g_ref[bb, :, pl.ds(k * PK, PK)]
        mads.append(jax.lax.dot_general(
            m2k, tblg_ref[bb], (((1,), (0,)), ((), ())),
            preferred_element_type=jnp.float32))
    madc = jnp.stack(mads, axis=0)
    madi = madc.transpose(1, 0, 2)
    mad_ref[...] = madi.reshape(T, N, 1, 1, G, C)


@functools.partial(jax.jit, static_argnames=())
def _run(pf_tn, wall, bias, ef, sx, sy, whtb):
    grid = (B // G, K2)
    out_shapes = (
        jax.ShapeDtypeStruct((T, N, B // G, G, C), jnp.float32),
        jax.ShapeDtypeStruct((T, N, K2, B // G, G, C), jnp.float32),
    )
    return pl.pallas_call(
        _dyn_kernel,
        grid=grid,
        in_specs=[
            pl.BlockSpec((T, N, G, C), lambda g, k: (0, 0, g, 0)),
            pl.BlockSpec((C, 2 * K2 * NCONV), lambda g, k: (0, 0)),
            pl.BlockSpec((1, 2 * NCONV), lambda g, k: (0, 0)),
            pl.BlockSpec((K2, 144), lambda g, k: (0, 0)),
            pl.BlockSpec((144, K2 * PK), lambda g, k: (0, 0)),
            pl.BlockSpec((144, K2 * PK), lambda g, k: (0, 0)),
            pl.BlockSpec((C, C), lambda g, k: (0, 0)),
        ],
        out_specs=(
            pl.BlockSpec((T, N, 1, G, C), lambda g, k: (0, 0, g, 0, 0)),
            pl.BlockSpec((T, N, 1, 1, G, C), lambda g, k: (0, 0, k, g, 0, 0)),
        ),
        out_shape=out_shapes,
        scratch_shapes=[
            pltpu.VMEM((G, TN, K2 * PK), BF),
            pltpu.VMEM((G, PK, C), BF),
            pltpu.VMEM((RG + 2 * MARG, 2 * K2 * NCONV), jnp.float32),
        ],
        compiler_params=pltpu.CompilerParams(
            dimension_semantics=("arbitrary", "arbitrary"),
        ),
    )(pf_tn, wall, bias, ef, sx, sy, whtb)


def kernel(person_features, W_hidden, Wp_1, bp_1, Ws_1, bs_1, Wp_2, bp_2, Ws_2, bs_2):
    pf_tn = person_features.transpose(1, 2, 0, 3)

    walls = []
    biases = []
    for Wp, bp, Ws, bs in ((Wp_1, bp_1, Ws_1, bs_1), (Wp_2, bp_2, Ws_2, bs_2)):
        wcat = jnp.concatenate([Wp, Ws], axis=0)
        w = wcat.transpose(2, 3, 1, 0).reshape(K2, C, NCONV)
        walls.append(w.transpose(1, 0, 2).reshape(C, K2 * NCONV))
        biases.append(jnp.concatenate([bp, bs], axis=0))
    wall = jnp.concatenate(walls, axis=1)
    bias = jnp.concatenate(biases, axis=0).reshape(1, 2 * NCONV)

    k_ar = np.arange(K2)
    ef_np = np.zeros((K2, 144), np.float32)
    ef_np[np.repeat(k_ar, 16), np.arange(144)] = 1.0
    sx_np = np.zeros((144, K2 * PK), np.float32)
    sy_np = np.zeros((144, K2 * PK), np.float32)
    for k in range(K2):
        for xx in range(TP):
            for yy in range(NP):
                p = k * PK + xx * NP + yy
                sx_np[k * 16 + xx, p] = 1.0
                sy_np[k * 16 + yy, p] = 1.0
    ef = jnp.asarray(ef_np, dtype=BF)
    sx = jnp.asarray(sx_np, dtype=BF)
    sy = jnp.asarray(sy_np, dtype=BF)

    whtb = W_hidden.T.astype(BF)

    dyn_p, mad_p = _run(pf_tn, wall, bias, ef, sx, sy, whtb)
    dyn = dyn_p.reshape(T, N, B, C).transpose(2, 0, 1, 3)
    mad = mad_p.reshape(T, N, K2, B, C).transpose(3, 0, 1, 2, 4)
    return dyn, mad


# speedup vs baseline: 3.1876x; 1.0030x over previous
import functools

import jax
import jax.numpy as jnp
import numpy as np
from jax.experimental import pallas as pl
from jax.experimental.pallas import tpu as pltpu

B, T, N, C = 64, 10, 12, 1024
K2 = 9
TN = T * N
G = 8
RG = TN * G
TP, NP = T + 4, N + 4
P = TP * NP
PK = 256
NCONV = 27
MARG = 26 * G
RATIOS = (1, 2)
BF = jnp.bfloat16


def _dyn_kernel(pf_ref, wall_ref, bias_ref, ef_ref, sx_ref, sy_ref, whtb_ref,
                dyn_ref, mad_ref, m2g_ref, tblg_ref, vp_ref):
    g = pl.program_id(0)
    k = pl.program_id(1)

    @pl.when((g == 0) & (k == 0))
    def _init():
        tblg_ref[...] = jnp.zeros_like(tblg_ref)
        vp_ref[...] = jnp.zeros_like(vp_ref)

    @pl.when(k == 0)
    def _frontend():
        xg = pf_ref[...]
        xi = xg.reshape(RG, C)

        xc = xi.astype(BF).reshape(TN, G, C).transpose(1, 0, 2)
        for t in range(T):
            tblg_ref[:, (t + 2) * NP + 2:(t + 2) * NP + 2 + N, :] = \
                xc[:, t * N:(t + 1) * N, :]

        v = jax.lax.dot_general(xi, wall_ref[...], (((1,), (0,)), ((), ())),
                                preferred_element_type=jnp.float32)
        vp_ref[MARG:MARG + RG, :] = v

        nrow = (jax.lax.broadcasted_iota(jnp.int32, (RG, 1), 0) // G) % N
        i144 = (jax.lax.broadcasted_iota(jnp.int32, (1, 144), 1) % 16
                ).astype(BF)

        ms_acc = None
        for r_idx, r in enumerate(RATIOS):
            acc = jnp.broadcast_to(
                bias_ref[0:1, r_idx * NCONV:(r_idx + 1) * NCONV],
                (RG, NCONV)).astype(jnp.float32)
            for kk_ in range(K2):
                di = (kk_ // 3 - 1) * r
                dj = (kk_ % 3 - 1) * r
                s = (di * N + dj) * G
                c0 = (r_idx * K2 + kk_) * NCONV
                sl = vp_ref[MARG + s:MARG + s + RG, c0:c0 + NCONV]
                nv = nrow + dj
                m = (nv >= 0) & (nv < N)
                acc = acc + jnp.where(m, sl, 0.0)

            offs = acc[:, :2 * K2]
            logits = acc[:, 2 * K2:NCONV]
            lmax = jnp.max(logits, axis=1, keepdims=True)
            e = jnp.exp(logits - lmax)
            scale = e / jnp.sum(e, axis=1, keepdims=True)

            rho = jax.lax.broadcasted_iota(jnp.int32, (RG, K2), 0)
            tt = (rho // (N * G)).astype(jnp.float32)
            nn = ((rho // G) % N).astype(jnp.float32)
            kk = jax.lax.broadcasted_iota(jnp.int32, (RG, K2), 1)
            tapx = ((kk // 3) - 1).astype(jnp.float32) * r
            tapy = ((kk % 3) - 1).astype(jnp.float32) * r
            pos_x = tt + r + tapx + offs[:, :K2]
            pos_y = nn + r + tapy + offs[:, K2:2 * K2]
            xmax = float(T + 2 * r - 1)
            ymax = float(N + 2 * r - 1)
            xl = jnp.clip(jnp.floor(pos_x), 0.0, xmax)
            xr = jnp.clip(jnp.floor(pos_x) + 1.0, 0.0, xmax)
            yl = jnp.clip(jnp.floor(pos_y), 0.0, ymax)
            yr = jnp.clip(jnp.floor(pos_y) + 1.0, 0.0, ymax)
            pxc = jnp.clip(pos_x, 0.0, xmax)
            pyc = jnp.clip(pos_y, 0.0, ymax)
            fs = float(2 - r)
            fields = jnp.concatenate(
                [1.0 - jnp.abs(pxc - xl), 1.0 - jnp.abs(pxc - xr),
                 xl + fs, xr + fs,
                 1.0 - jnp.abs(pyc - yl), 1.0 - jnp.abs(pyc - yr),
                 yl + fs, yr + fs],
                axis=0).astype(BF)

            bc = jax.lax.dot_general(fields, ef_ref[...],
                                     (((1,), (0,)), ((), ())),
                                     preferred_element_type=jnp.float32)
            bcb = bc.astype(BF)
            axl = jnp.where(i144 == bcb[2 * RG:3 * RG], bcb[0:RG], 0.0) + \
                jnp.where(i144 == bcb[3 * RG:4 * RG], bcb[RG:2 * RG], 0.0)
            ayl = jnp.where(i144 == bcb[6 * RG:7 * RG], bcb[4 * RG:5 * RG], 0.0) + \
                jnp.where(i144 == bcb[7 * RG:8 * RG], bcb[5 * RG:6 * RG], 0.0)

            m2parts = []
            for c0 in range(0, K2 * PK, 3 * PK):
                axv = jax.lax.dot_general(
                    axl, sx_ref[:, c0:c0 + 3 * PK], (((1,), (0,)), ((), ())),
                    preferred_element_type=jnp.float32)
                ayv = jax.lax.dot_general(
                    ayl, sy_ref[:, c0:c0 + 3 * PK], (((1,), (0,)), ((), ())),
                    preferred_element_type=jnp.float32)
                m2parts.append((axv * ayv).astype(BF))
            m2l = jnp.concatenate(m2parts, axis=1)

            msr = None
            for kk_ in range(K2):
                term = scale[:, kk_:kk_ + 1] * \
                    m2l[:, kk_ * PK:(kk_ + 1) * PK].astype(jnp.float32)
                msr = term if msr is None else msr + term
            ms_acc = msr if ms_acc is None else ms_acc + msr

            if r == 2:
                m2g_ref[...] = m2l.reshape(TN, G, K2 * PK).transpose(1, 0, 2)

        msb = (ms_acc * 0.5).astype(BF).reshape(TN, G, PK).transpose(1, 0, 2)
        ftms = []
        for bb in range(G):
            ftms.append(jax.lax.dot_general(
                msb[bb], tblg_ref[bb], (((1,), (0,)), ((), ())),
                preferred_element_type=jnp.float32).astype(BF))
        ftmc = jnp.concatenate(ftms, axis=0)
        dync = jax.lax.dot_general(ftmc, whtb_ref[...],
                                   (((1,), (0,)), ((), ())),
                                   preferred_element_type=jnp.float32)
        dyn = dync.reshape(G, TN, C).transpose(1, 0, 2)
        dyn_ref[...] = dyn.reshape(T, N, 1, G, C)

    mads = []
    for bb in range(G):
        m2k = m2g_ref[bb, :, pl.ds(k * PK, PK)]
        mads.append(jax.lax.dot_general(
            m2k, tblg_ref[bb], (((1,), (0,)), ((), ())),
            preferred_element_type=jnp.float32))
    madc = jnp.stack(mads, axis=0)
    madi = madc.transpose(1, 0, 2)
    mad_ref[...] = madi.reshape(T, N, 1, 1, G, C)


@functools.partial(jax.jit, static_argnames=())
def _run(pf_tn, wall, bias, ef, sx, sy, whtb):
    grid = (B // G, K2)
    out_shapes = (
        jax.ShapeDtypeStruct((T, N, B // G, G, C), jnp.float32),
        jax.ShapeDtypeStruct((T, N, K2, B // G, G, C), jnp.float32),
    )
    return pl.pallas_call(
        _dyn_kernel,
        grid=grid,
        in_specs=[
            pl.BlockSpec((T, N, G, C), lambda g, k: (0, 0, g, 0)),
            pl.BlockSpec((C, 2 * K2 * NCONV), lambda g, k: (0, 0)),
            pl.BlockSpec((1, 2 * NCONV), lambda g, k: (0, 0)),
            pl.BlockSpec((K2, 144), lambda g, k: (0, 0)),
            pl.BlockSpec((144, K2 * PK), lambda g, k: (0, 0)),
            pl.BlockSpec((144, K2 * PK), lambda g, k: (0, 0)),
            pl.BlockSpec((C, C), lambda g, k: (0, 0)),
        ],
        out_specs=(
            pl.BlockSpec((T, N, 1, G, C), lambda g, k: (0, 0, g, 0, 0)),
            pl.BlockSpec((T, N, 1, 1, G, C), lambda g, k: (0, 0, k, g, 0, 0)),
        ),
        out_shape=out_shapes,
        scratch_shapes=[
            pltpu.VMEM((G, TN, K2 * PK), BF),
            pltpu.VMEM((G, PK, C), BF),
            pltpu.VMEM((RG + 2 * MARG, 2 * K2 * NCONV), jnp.float32),
        ],
        compiler_params=pltpu.CompilerParams(
            dimension_semantics=("arbitrary", "arbitrary"),
        ),
    )(pf_tn, wall, bias, ef, sx, sy, whtb)


def kernel(person_features, W_hidden, Wp_1, bp_1, Ws_1, bs_1, Wp_2, bp_2, Ws_2, bs_2):
    pf_tn = person_features.transpose(1, 2, 0, 3)

    walls = []
    biases = []
    for Wp, bp, Ws, bs in ((Wp_1, bp_1, Ws_1, bs_1), (Wp_2, bp_2, Ws_2, bs_2)):
        wcat = jnp.concatenate([Wp, Ws], axis=0)
        w = wcat.transpose(2, 3, 1, 0).reshape(K2, C, NCONV)
        walls.append(w.transpose(1, 0, 2).reshape(C, K2 * NCONV))
        biases.append(jnp.concatenate([bp, bs], axis=0))
    wall = jnp.concatenate(walls, axis=1)
    bias = jnp.concatenate(biases, axis=0).reshape(1, 2 * NCONV)

    k_ar = np.arange(K2)
    ef_np = np.zeros((K2, 144), np.float32)
    ef_np[np.repeat(k_ar, 16), np.arange(144)] = 1.0
    sx_np = np.zeros((144, K2 * PK), np.float32)
    sy_np = np.zeros((144, K2 * PK), np.float32)
    for k in range(K2):
        for xx in range(TP):
            for yy in range(NP):
                p = k * PK + xx * NP + yy
                sx_np[k * 16 + xx, p] = 1.0
                sy_np[k * 16 + yy, p] = 1.0
    ef = jnp.asarray(ef_np, dtype=BF)
    sx = jnp.asarray(sx_np, dtype=BF)
    sy = jnp.asarray(sy_np, dtype=BF)

    whtb = W_hidden.T.astype(BF)

    dyn_p, mad_p = _run(pf_tn, wall, bias, ef, sx, sy, whtb)
    dyn = dyn_p.reshape(T, N, B, C).transpose(2, 0, 1, 3)
    mad = mad_p.reshape(T, N, K2, B, C).transpose(3, 0, 1, 2, 4)
    return dyn, mad
